# baseline (speedup 1.0000x reference)
import jax
import jax.numpy as jnp
from jax.experimental import pallas as pl
from jax.experimental.pallas import tpu as pltpu


def _gather_body(cols_ref, x_ref, o_ref):
    jc = x_ref.shape[1]
    kc = o_ref.shape[1]
    cols = cols_ref[0, :]
    rows = jax.lax.broadcasted_iota(jnp.int32, (jc, kc), 0)
    sel = jnp.where(rows == cols[None, :], 1.0, 0.0).astype(jnp.float32)
    o_ref[...] = jnp.dot(x_ref[...], sel, preferred_element_type=jnp.float32)


def kernel(joints, joint_maps):
    b, j, c = joints.shape
    k = joint_maps.shape[0]
    x = joints.reshape(b, j * c)
    cols = (joint_maps[:, None] * c
            + jnp.arange(c, dtype=jnp.int32)[None, :]).reshape(1, k * c)
    blk = 2048
    out = pl.pallas_call(
        _gather_body,
        grid=(b // blk,),
        in_specs=[
            pl.BlockSpec((1, k * c), lambda i: (0, 0)),
            pl.BlockSpec((blk, j * c), lambda i: (i, 0)),
        ],
        out_specs=pl.BlockSpec((blk, k * c), lambda i: (i, 0)),
        out_shape=jax.ShapeDtypeStruct((b, k * c), jnp.float32),
    )(cols, x)
    return out.reshape(b, k, c)


# speedup vs baseline: 14.8563x; 14.8563x over previous
import jax
import jax.numpy as jnp
from jax.experimental import pallas as pl
from jax.experimental.pallas import tpu as pltpu


def _gather_body(maps_ref, x_ref, o_ref):
    c, j, _ = x_ref.shape
    k = o_ref.shape[1]
    maps = maps_ref[0, :]
    cols = jax.lax.broadcasted_iota(jnp.int32, (k, j), 1)
    sel = jnp.where(cols == maps[:, None], 1.0, 0.0).astype(jnp.float32)
    for ci in range(c):
        o_ref[ci] = jnp.dot(sel, x_ref[ci], preferred_element_type=jnp.float32)


def kernel(joints, joint_maps):
    b, j, c = joints.shape
    k = joint_maps.shape[0]
    xt = jnp.transpose(joints, (2, 1, 0))
    maps = joint_maps.reshape(1, k)
    blk = 2048
    out_t = pl.pallas_call(
        _gather_body,
        grid=(b // blk,),
        in_specs=[
            pl.BlockSpec((1, k), lambda i: (0, 0)),
            pl.BlockSpec((c, j, blk), lambda i: (0, 0, i)),
        ],
        out_specs=pl.BlockSpec((c, k, blk), lambda i: (0, 0, i)),
        out_shape=jax.ShapeDtypeStruct((c, k, b), jnp.float32),
    )(maps, xt)
    return jnp.transpose(out_t, (2, 1, 0))


# speedup vs baseline: 15.8609x; 1.0676x over previous
import jax
import jax.numpy as jnp
from jax.experimental import pallas as pl
from jax.experimental.pallas import tpu as pltpu


def _gather_body(maps_ref, x_ref, o_ref):
    c, j, _ = x_ref.shape
    k = o_ref.shape[1]
    maps = maps_ref[0, :]
    cols = jax.lax.broadcasted_iota(jnp.int32, (k, j), 1)
    sel = jnp.where(cols == maps[:, None], 1.0, 0.0).astype(jnp.float32)
    for ci in range(c):
        o_ref[ci] = jnp.dot(sel, x_ref[ci], preferred_element_type=jnp.float32)


def kernel(joints, joint_maps):
    b, j, c = joints.shape
    k = joint_maps.shape[0]
    xt = jnp.transpose(joints, (2, 1, 0))
    maps = joint_maps.reshape(1, k)
    blk = 4096
    out_t = pl.pallas_call(
        _gather_body,
        grid=(b // blk,),
        in_specs=[
            pl.BlockSpec((1, k), lambda i: (0, 0)),
            pl.BlockSpec((c, j, blk), lambda i: (0, 0, i)),
        ],
        out_specs=pl.BlockSpec((c, k, blk), lambda i: (0, 0, i)),
        out_shape=jax.ShapeDtypeStruct((c, k, b), jnp.float32),
    )(maps, xt)
    return jnp.transpose(out_t, (2, 1, 0))


# speedup vs baseline: 17.3269x; 1.0924x over previous
import jax
import jax.numpy as jnp
from jax.experimental import pallas as pl
from jax.experimental.pallas import tpu as pltpu


def _gather_body(maps_ref, x_ref, o_ref):
    c, j, _ = x_ref.shape
    k = o_ref.shape[1]
    maps = maps_ref[0, :]
    cols = jax.lax.broadcasted_iota(jnp.int32, (k, j), 1)
    sel = jnp.where(cols == maps[:, None], 1.0, 0.0).astype(jnp.float32)
    for ci in range(c):
        o_ref[ci] = jnp.dot(sel, x_ref[ci], preferred_element_type=jnp.float32)


def kernel(joints, joint_maps):
    b, j, c = joints.shape
    k = joint_maps.shape[0]
    xt = jnp.transpose(joints, (2, 1, 0))
    maps = joint_maps.reshape(1, k)
    blk = 8192
    out_t = pl.pallas_call(
        _gather_body,
        grid=(b // blk,),
        in_specs=[
            pl.BlockSpec((1, k), lambda i: (0, 0)),
            pl.BlockSpec((c, j, blk), lambda i: (0, 0, i)),
        ],
        out_specs=pl.BlockSpec((c, k, blk), lambda i: (0, 0, i)),
        out_shape=jax.ShapeDtypeStruct((c, k, b), jnp.float32),
    )(maps, xt)
    return jnp.transpose(out_t, (2, 1, 0))
